# Initial kernel scaffold; baseline (speedup 1.0000x reference)
import jax
import jax.numpy as jnp
from jax.experimental import pallas as pl


def kernel(node_features, edge_index, edge_norm, edge_type, basis, comp, rgcn_root, rgcn_bias, gc_w_rel, gc_b_rel, gc_w_root):
    raise NotImplementedError("write your pallas kernel here")


# speedup vs baseline: 6.9537x; 6.9537x over previous
import functools

import jax
import jax.numpy as jnp
from jax import lax
from jax.experimental import pallas as pl
from jax.experimental.pallas import tpu as pltpu
from jax.experimental.pallas import tpu_sc as plsc

N = 10000
E = 320000
R = 4
NB = 30
G = 128
H = 128

NC = 2
NS = 16
NW = NC * NS
L = 16

CH = 128
EPW = 10112
NCHUNK = EPW // CH
EPAD = NW * EPW

NRP = 40960
CNTW = 16
NP = 10112

_MESH = plsc.VectorSubcoreMesh(
    core_axis_name="c", subcore_axis_name="s", num_cores=NC, num_subcores=NS)


def _wid():
    return lax.axis_index("s") * NC + lax.axis_index("c")


@functools.partial(
    pl.kernel,
    out_type=jax.ShapeDtypeStruct((NC * NRP,), jnp.float32),
    mesh=_MESH,
    compiler_params=pltpu.CompilerParams(needs_layout_passes=False),
    scratch_types=[
        pltpu.VMEM((CH,), jnp.int32),
        pltpu.VMEM((CH,), jnp.int32),
        pltpu.VMEM((CH,), jnp.int32),
        pltpu.VMEM((CH,), jnp.float32),
        pltpu.VMEM_SHARED((NRP,), jnp.float32),
    ],
)
def _sc_counts(dst_hbm, typ_hbm, zc_hbm, ones_hbm, out_hbm,
               dstv, typv, segv, onesv, cnt_sh):
    cid = lax.axis_index("c")
    sid = lax.axis_index("s")
    wid = _wid()
    sl = NRP // NS
    pltpu.sync_copy(zc_hbm.at[pl.ds(sid * sl, sl)], cnt_sh.at[pl.ds(sid * sl, sl)])
    pltpu.sync_copy(ones_hbm, onesv)
    plsc.subcore_barrier()

    def chunk(c, carry):
        base = pl.multiple_of(wid * EPW + c * CH, CH)
        pltpu.sync_copy(dst_hbm.at[pl.ds(base, CH)], dstv)
        pltpu.sync_copy(typ_hbm.at[pl.ds(base, CH)], typv)
        for j in range(CH // L):
            d16 = dstv[pl.ds(j * L, L)]
            t16 = typv[pl.ds(j * L, L)]
            segv[pl.ds(j * L, L)] = d16 * R + t16
        pltpu.sync_copy(onesv, cnt_sh.at[segv], add=True)
        return carry

    lax.fori_loop(0, NCHUNK, chunk, 0)
    plsc.subcore_barrier()
    pltpu.sync_copy(cnt_sh.at[pl.ds(sid * sl, sl)],
                    out_hbm.at[pl.ds(cid * NRP + sid * sl, sl)])


@functools.partial(
    pl.kernel,
    out_type=jax.ShapeDtypeStruct((NC, NP, H), jnp.float32),
    mesh=_MESH,
    compiler_params=pltpu.CompilerParams(needs_layout_passes=False),
    scratch_types=[
        pltpu.VMEM((CH,), jnp.int32),
        pltpu.VMEM((CH,), jnp.int32),
        pltpu.VMEM((CH,), jnp.int32),
        pltpu.VMEM((CH,), jnp.int32),
        pltpu.VMEM((CH,), jnp.int32),
        pltpu.VMEM((CH,), jnp.float32),
        pltpu.VMEM((CH, H), jnp.float32),
        pltpu.VMEM_SHARED((NP, H), jnp.float32),
        pltpu.SemaphoreType.DMA,
        pltpu.SemaphoreType.DMA,
    ],
)
def _sc_rgcn(xw_hbm, inv_hbm, src_hbm, dst_hbm, typ_hbm, zr_hbm, out_hbm,
             srcv, dstv, typv, gidxv, segv, sv, rows, acc, sem, sem2):
    cid = lax.axis_index("c")
    sid = lax.axis_index("s")
    wid = _wid()
    zsl = NP // NS
    pltpu.sync_copy(zr_hbm.at[pl.ds(sid * zsl, zsl)], acc.at[pl.ds(sid * zsl, zsl)])
    plsc.subcore_barrier()

    def chunk(c, carry):
        base = pl.multiple_of(wid * EPW + c * CH, CH)
        pltpu.sync_copy(src_hbm.at[pl.ds(base, CH)], srcv)
        pltpu.sync_copy(dst_hbm.at[pl.ds(base, CH)], dstv)
        pltpu.sync_copy(typ_hbm.at[pl.ds(base, CH)], typv)
        for j in range(CH // L):
            s16 = srcv[pl.ds(j * L, L)]
            t16 = typv[pl.ds(j * L, L)]
            d16 = dstv[pl.ds(j * L, L)]
            gidxv[pl.ds(j * L, L)] = t16 * N + s16
            segv[pl.ds(j * L, L)] = d16 * R + t16
        cp = pltpu.async_copy(xw_hbm.at[gidxv], rows, sem)
        cps = pltpu.async_copy(inv_hbm.at[segv], sv, sem2)
        cps.wait()
        cp.wait()

        def escale(e, cy):
            ssp = plsc.load_gather(sv, [lax.broadcast(e, (L,))])
            for db in range(H // L):
                rows[e, pl.ds(db * L, L)] = rows[e, pl.ds(db * L, L)] * ssp
            return cy

        lax.fori_loop(0, CH, escale, 0)
        pltpu.sync_copy(rows, acc.at[dstv], add=True)
        return carry

    lax.fori_loop(0, NCHUNK, chunk, 0)
    plsc.subcore_barrier()
    osl = NP // NS
    pltpu.sync_copy(acc.at[pl.ds(sid * osl, osl)],
                    out_hbm.at[cid, pl.ds(sid * osl, osl)])


@functools.partial(
    pl.kernel,
    out_type=jax.ShapeDtypeStruct((NC, NP, H), jnp.float32),
    mesh=_MESH,
    compiler_params=pltpu.CompilerParams(needs_layout_passes=False),
    scratch_types=[
        pltpu.VMEM((CH,), jnp.int32),
        pltpu.VMEM((CH,), jnp.int32),
        pltpu.VMEM((CH, H), jnp.float32),
        pltpu.VMEM_SHARED((NP, H), jnp.float32),
        pltpu.SemaphoreType.DMA,
    ],
)
def _sc_gconv(h_hbm, src_hbm, dst_hbm, zr_hbm, out_hbm, srcv, dstv, rows, acc, sem):
    cid = lax.axis_index("c")
    sid = lax.axis_index("s")
    wid = _wid()
    zsl = NP // NS
    pltpu.sync_copy(zr_hbm.at[pl.ds(sid * zsl, zsl)], acc.at[pl.ds(sid * zsl, zsl)])
    plsc.subcore_barrier()

    def chunk(c, carry):
        base = pl.multiple_of(wid * EPW + c * CH, CH)
        pltpu.sync_copy(src_hbm.at[pl.ds(base, CH)], srcv)
        pltpu.sync_copy(dst_hbm.at[pl.ds(base, CH)], dstv)
        pltpu.async_copy(h_hbm.at[srcv], rows, sem).wait()
        pltpu.sync_copy(rows, acc.at[dstv], add=True)
        return carry

    lax.fori_loop(0, NCHUNK, chunk, 0)
    plsc.subcore_barrier()
    osl = NP // NS
    pltpu.sync_copy(acc.at[pl.ds(sid * osl, osl)],
                    out_hbm.at[cid, pl.ds(sid * osl, osl)])


def _tc_weight_body(comp_ref, basis_ref, out_ref):
    out_ref[...] = jnp.dot(comp_ref[...], basis_ref[...],
                           preferred_element_type=jnp.float32)


def _tc_weight(comp, basis2):
    return pl.pallas_call(
        _tc_weight_body,
        out_shape=jax.ShapeDtypeStruct((R, G * H), jnp.float32),
    )(comp, basis2)


def _tc_inv_body(cnt_ref, out_ref):
    c = cnt_ref[0] + cnt_ref[1]
    out_ref[...] = (1.0 / jnp.maximum(c, 1.0))[None, :]


def _tc_inv(cnt_parts):
    return pl.pallas_call(
        _tc_inv_body,
        out_shape=jax.ShapeDtypeStruct((1, NRP), jnp.float32),
    )(cnt_parts)


BN = 400
NBLK = N // BN


def _tc_xw_body(x_ref, w_ref, out_ref):
    out_ref[...] = jnp.dot(x_ref[...], w_ref[0],
                           preferred_element_type=jnp.float32)


def _tc_xw(x, w3):
    return pl.pallas_call(
        _tc_xw_body,
        grid=(R, NBLK),
        in_specs=[
            pl.BlockSpec((BN, G), lambda r, i: (i, 0)),
            pl.BlockSpec((1, G, H), lambda r, i: (r, 0, 0)),
        ],
        out_specs=pl.BlockSpec((BN, H), lambda r, i: (r * NBLK + i, 0)),
        out_shape=jax.ShapeDtypeStruct((R * N, H), jnp.float32),
    )(x, w3)


def _tc_h_body(parts_ref, x_ref, root_ref, bias_ref, out_ref):
    p = parts_ref[...]
    out_ref[...] = (p[0] + p[1]
                    + jnp.dot(x_ref[...], root_ref[...],
                              preferred_element_type=jnp.float32)
                    + bias_ref[...])


def _tc_h(parts, x, root, bias2):
    return pl.pallas_call(
        _tc_h_body,
        grid=(NBLK,),
        in_specs=[
            pl.BlockSpec((NC, BN, H), lambda i: (0, i, 0)),
            pl.BlockSpec((BN, G), lambda i: (i, 0)),
            pl.BlockSpec((G, H), lambda i: (0, 0)),
            pl.BlockSpec((1, H), lambda i: (0, 0)),
        ],
        out_specs=pl.BlockSpec((BN, H), lambda i: (i, 0)),
        out_shape=jax.ShapeDtypeStruct((N, H), jnp.float32),
    )(parts, x, root, bias2)


def _tc_out_body(parts_ref, h_ref, wrel_ref, brel_ref, wroot_ref, out_ref):
    p = parts_ref[...]
    out_ref[...] = (jnp.dot(p[0] + p[1], wrel_ref[...],
                            preferred_element_type=jnp.float32)
                    + brel_ref[...]
                    + jnp.dot(h_ref[...], wroot_ref[...],
                              preferred_element_type=jnp.float32))


def _tc_out(parts, h, wrel, brel2, wroot):
    return pl.pallas_call(
        _tc_out_body,
        grid=(NBLK,),
        in_specs=[
            pl.BlockSpec((NC, BN, H), lambda i: (0, i, 0)),
            pl.BlockSpec((BN, H), lambda i: (i, 0)),
            pl.BlockSpec((H, H), lambda i: (0, 0)),
            pl.BlockSpec((1, H), lambda i: (0, 0)),
            pl.BlockSpec((H, H), lambda i: (0, 0)),
        ],
        out_specs=pl.BlockSpec((BN, H), lambda i: (i, 0)),
        out_shape=jax.ShapeDtypeStruct((N, H), jnp.float32),
    )(parts, h, wrel, brel2, wroot)


def kernel(node_features, edge_index, edge_norm, edge_type, basis, comp,
           rgcn_root, rgcn_bias, gc_w_rel, gc_b_rel, gc_w_root):
    del edge_norm
    src = edge_index[0]
    dst = edge_index[1]
    pad = EPAD - E
    src_p = jnp.concatenate([src, jnp.zeros((pad,), jnp.int32)])
    dst_p = jnp.concatenate([dst, jnp.full((pad,), N, jnp.int32)])
    typ_p = jnp.concatenate([edge_type, jnp.zeros((pad,), jnp.int32)])
    zeros_cnt = jnp.zeros((NRP,), jnp.float32)
    zeros_rows = jnp.zeros((NP, H), jnp.float32)
    ones_tpl = jnp.ones((CH,), jnp.float32)

    cnt_parts = _sc_counts(dst_p, typ_p, zeros_cnt, ones_tpl).reshape(NC, NRP)
    inv = _tc_inv(cnt_parts).reshape(NRP)

    w3 = _tc_weight(comp, basis.reshape(NB, G * H)).reshape(R, G, H)
    xw = _tc_xw(node_features, w3)

    h_parts = _sc_rgcn(xw, inv, src_p, dst_p, typ_p, zeros_rows)
    h = _tc_h(h_parts, node_features, rgcn_root, rgcn_bias.reshape(1, H))

    agg_parts = _sc_gconv(h, src_p, dst_p, zeros_rows)
    out = _tc_out(agg_parts, h, gc_w_rel, gc_b_rel.reshape(1, H), gc_w_root)
    return out


# speedup vs baseline: 7.3802x; 1.0613x over previous
import functools

import jax
import jax.numpy as jnp
from jax import lax
from jax.experimental import pallas as pl
from jax.experimental.pallas import tpu as pltpu
from jax.experimental.pallas import tpu_sc as plsc

N = 10000
E = 320000
R = 4
NB = 30
G = 128
H = 128

NC = 2
NS = 16
NW = NC * NS
L = 16

CH = 128
ERW = E // NW
NCHUNK = -(-ERW // CH)
NCH2 = NCHUNK + 2
EPWD = NCH2 * CH

NRP = 40960
NP = 10112

_MESH = plsc.VectorSubcoreMesh(
    core_axis_name="c", subcore_axis_name="s", num_cores=NC, num_subcores=NS)
_SC_PARAMS = pltpu.CompilerParams(needs_layout_passes=False)


def _wid():
    return lax.axis_index("s") * NC + lax.axis_index("c")


def _echunk(edata_hbm, wid, c):
    off = pl.multiple_of((wid * NCH2 + c) * (3 * CH), CH)
    return edata_hbm.at[pl.ds(off, 3 * CH)]


@functools.partial(
    pl.kernel,
    out_type=jax.ShapeDtypeStruct((NC * NRP,), jnp.float32),
    mesh=_MESH,
    compiler_params=_SC_PARAMS,
    scratch_types=[
        pltpu.VMEM((2, 3 * CH), jnp.int32),
        pltpu.VMEM((2, CH), jnp.int32),
        pltpu.VMEM((CH,), jnp.float32),
        pltpu.VMEM_SHARED((NRP,), jnp.float32),
        pltpu.SemaphoreType.DMA,
        pltpu.SemaphoreType.DMA,
    ],
)
def _sc_counts(edata_hbm, zc_hbm, ones_hbm, out_hbm,
               ebuf, segv, onesv, cnt_sh, es0, es1):
    es = (es0, es1)
    cid = lax.axis_index("c")
    sid = lax.axis_index("s")
    wid = _wid()
    sl = NRP // NS
    pltpu.sync_copy(zc_hbm.at[pl.ds(sid * sl, sl)], cnt_sh.at[pl.ds(sid * sl, sl)])
    pltpu.sync_copy(ones_hbm, onesv)

    def e_issue(c, b):
        pltpu.async_copy(_echunk(edata_hbm, wid, c), ebuf.at[b], es[b])

    def e_wait(c, b):
        pltpu.make_async_copy(_echunk(edata_hbm, wid, c), ebuf.at[b], es[b]).wait()

    def seg_compute(b):
        for j in range(CH // L):
            d16 = ebuf[b, pl.ds(CH + j * L, L)]
            t16 = ebuf[b, pl.ds(2 * CH + j * L, L)]
            segv[b, pl.ds(j * L, L)] = d16 * R + t16

    def scat(b):
        pltpu.sync_copy(onesv, cnt_sh.at[segv.at[b]], add=True)

    plsc.subcore_barrier()
    e_issue(0, 0)
    e_issue(1, 1)
    e_wait(0, 0)
    seg_compute(0)
    e_issue(2, 0)
    scat(0)

    def pair(p, carry):
        for off, b in ((1, 1), (2, 0)):
            c = 2 * p + off
            e_wait(c, b)
            seg_compute(b)
            e_issue(c + 2, b)
            scat(b)
        return carry

    lax.fori_loop(0, (NCHUNK - 1) // 2, pair, 0)
    e_wait(NCH2 - 2, 1)
    e_wait(NCH2 - 1, 0)
    plsc.subcore_barrier()
    pltpu.sync_copy(cnt_sh.at[pl.ds(sid * sl, sl)],
                    out_hbm.at[pl.ds(cid * NRP + sid * sl, sl)])


@functools.partial(
    pl.kernel,
    out_type=jax.ShapeDtypeStruct((NC, NP, H), jnp.float32),
    mesh=_MESH,
    compiler_params=_SC_PARAMS,
    scratch_types=[
        pltpu.VMEM((2, 3 * CH), jnp.int32),
        pltpu.VMEM((2, CH), jnp.int32),
        pltpu.VMEM((2, CH), jnp.int32),
        pltpu.VMEM((2, CH), jnp.int32),
        pltpu.VMEM((2 * CH,), jnp.float32),
        pltpu.VMEM((2, CH, H), jnp.float32),
        pltpu.VMEM_SHARED((NP, H), jnp.float32),
        pltpu.SemaphoreType.DMA,
        pltpu.SemaphoreType.DMA,
        pltpu.SemaphoreType.DMA,
        pltpu.SemaphoreType.DMA,
        pltpu.SemaphoreType.DMA,
        pltpu.SemaphoreType.DMA,
        pltpu.SemaphoreType.DMA,
        pltpu.SemaphoreType.DMA,
    ],
)
def _sc_rgcn(xw_hbm, inv_hbm, edata_hbm, zr_hbm, out_hbm,
             ebuf, gidxv, segv, dstv, sv, rows, acc,
             es0, es1, gs0, gs1, ss0, ss1, vs0, vs1):
    es = (es0, es1)
    gs = (gs0, gs1)
    ss = (ss0, ss1)
    vs = (vs0, vs1)
    cid = lax.axis_index("c")
    sid = lax.axis_index("s")
    wid = _wid()
    zsl = NP // NS
    pltpu.sync_copy(zr_hbm.at[pl.ds(sid * zsl, zsl)], acc.at[pl.ds(sid * zsl, zsl)])

    def e_issue(c, b):
        pltpu.async_copy(_echunk(edata_hbm, wid, c), ebuf.at[b], es[b])

    def e_wait(c, b):
        pltpu.make_async_copy(_echunk(edata_hbm, wid, c), ebuf.at[b], es[b]).wait()

    def prep(b):
        for j in range(CH // L):
            s16 = ebuf[b, pl.ds(j * L, L)]
            d16 = ebuf[b, pl.ds(CH + j * L, L)]
            t16 = ebuf[b, pl.ds(2 * CH + j * L, L)]
            gidxv[b, pl.ds(j * L, L)] = t16 * N + s16
            segv[b, pl.ds(j * L, L)] = d16 * R + t16
            dstv[b, pl.ds(j * L, L)] = d16
        pltpu.async_copy(xw_hbm.at[gidxv.at[b]], rows.at[b], gs[b])
        pltpu.async_copy(inv_hbm.at[segv.at[b]], sv.at[pl.ds(b * CH, CH)], vs[b])

    def g_wait(b):
        pltpu.make_async_copy(xw_hbm.at[gidxv.at[b]], rows.at[b], gs[b]).wait()
        pltpu.make_async_copy(inv_hbm.at[segv.at[b]],
                              sv.at[pl.ds(b * CH, CH)], vs[b]).wait()

    def scale(b):
        def body(i, carry):
            for u in range(2):
                e = i * 2 + u
                ssp = plsc.load_gather(sv, [jnp.broadcast_to(b * CH + e, (L,))])
                for db in range(H // L):
                    rows[b, e, pl.ds(db * L, L)] = (
                        rows[b, e, pl.ds(db * L, L)] * ssp)
            return carry
        lax.fori_loop(0, CH // 2, body, 0)

    def s_issue(b):
        pltpu.async_copy(rows.at[b], acc.at[dstv.at[b]], ss[b], add=True)

    def s_wait(b):
        pltpu.make_async_copy(rows.at[b], acc.at[dstv.at[b]], ss[b]).wait()

    plsc.subcore_barrier()
    e_issue(0, 0)
    e_issue(1, 1)
    e_wait(0, 0)
    prep(0)
    g_wait(0)
    scale(0)
    s_issue(0)
    e_issue(2, 0)
    e_wait(1, 1)
    prep(1)

    def pair(p, carry):
        for off, b in ((1, 1), (2, 0)):
            c = 2 * p + off
            b2 = 1 - b
            g_wait(b)
            scale(b)
            s_issue(b)
            e_issue(c + 2, b)
            s_wait(b2)
            e_wait(c + 1, b2)
            prep(b2)
        return carry

    lax.fori_loop(0, (NCHUNK - 1) // 2, pair, 0)
    s_wait(0)
    g_wait(1)
    e_wait(NCH2 - 1, 0)
    plsc.subcore_barrier()
    osl = NP // NS
    pltpu.sync_copy(acc.at[pl.ds(sid * osl, osl)],
                    out_hbm.at[cid, pl.ds(sid * osl, osl)])


@functools.partial(
    pl.kernel,
    out_type=jax.ShapeDtypeStruct((NC, NP, H), jnp.float32),
    mesh=_MESH,
    compiler_params=_SC_PARAMS,
    scratch_types=[
        pltpu.VMEM((2, 3 * CH), jnp.int32),
        pltpu.VMEM((2, CH), jnp.int32),
        pltpu.VMEM((2, CH), jnp.int32),
        pltpu.VMEM((2, CH, H), jnp.float32),
        pltpu.VMEM_SHARED((NP, H), jnp.float32),
        pltpu.SemaphoreType.DMA,
        pltpu.SemaphoreType.DMA,
        pltpu.SemaphoreType.DMA,
        pltpu.SemaphoreType.DMA,
        pltpu.SemaphoreType.DMA,
        pltpu.SemaphoreType.DMA,
    ],
)
def _sc_gconv(h_hbm, edata_hbm, zr_hbm, out_hbm,
              ebuf, srcv, dstv, rows, acc, es0, es1, gs0, gs1, ss0, ss1):
    es = (es0, es1)
    gs = (gs0, gs1)
    ss = (ss0, ss1)
    cid = lax.axis_index("c")
    sid = lax.axis_index("s")
    wid = _wid()
    zsl = NP // NS
    pltpu.sync_copy(zr_hbm.at[pl.ds(sid * zsl, zsl)], acc.at[pl.ds(sid * zsl, zsl)])

    def e_issue(c, b):
        pltpu.async_copy(_echunk(edata_hbm, wid, c), ebuf.at[b], es[b])

    def e_wait(c, b):
        pltpu.make_async_copy(_echunk(edata_hbm, wid, c), ebuf.at[b], es[b]).wait()

    def prep(b):
        for j in range(CH // L):
            srcv[b, pl.ds(j * L, L)] = ebuf[b, pl.ds(j * L, L)]
            dstv[b, pl.ds(j * L, L)] = ebuf[b, pl.ds(CH + j * L, L)]
        pltpu.async_copy(h_hbm.at[srcv.at[b]], rows.at[b], gs[b])

    def g_wait(b):
        pltpu.make_async_copy(h_hbm.at[srcv.at[b]], rows.at[b], gs[b]).wait()

    def s_issue(b):
        pltpu.async_copy(rows.at[b], acc.at[dstv.at[b]], ss[b], add=True)

    def s_wait(b):
        pltpu.make_async_copy(rows.at[b], acc.at[dstv.at[b]], ss[b]).wait()

    plsc.subcore_barrier()
    e_issue(0, 0)
    e_issue(1, 1)
    e_wait(0, 0)
    prep(0)
    g_wait(0)
    s_issue(0)
    e_issue(2, 0)
    e_wait(1, 1)
    prep(1)

    def pair(p, carry):
        for off, b in ((1, 1), (2, 0)):
            c = 2 * p + off
            b2 = 1 - b
            g_wait(b)
            s_issue(b)
            e_issue(c + 2, b)
            s_wait(b2)
            e_wait(c + 1, b2)
            prep(b2)
        return carry

    lax.fori_loop(0, (NCHUNK - 1) // 2, pair, 0)
    s_wait(0)
    g_wait(1)
    e_wait(NCH2 - 1, 0)
    plsc.subcore_barrier()
    osl = NP // NS
    pltpu.sync_copy(acc.at[pl.ds(sid * osl, osl)],
                    out_hbm.at[cid, pl.ds(sid * osl, osl)])


def _tc_weight_body(comp_ref, basis_ref, out_ref):
    out_ref[...] = jnp.dot(comp_ref[...], basis_ref[...],
                           preferred_element_type=jnp.float32)


def _tc_weight(comp, basis2):
    return pl.pallas_call(
        _tc_weight_body,
        out_shape=jax.ShapeDtypeStruct((R, G * H), jnp.float32),
    )(comp, basis2)


def _tc_inv_body(cnt_ref, out_ref):
    c = cnt_ref[0] + cnt_ref[1]
    out_ref[...] = (1.0 / jnp.maximum(c, 1.0))[None, :]


def _tc_inv(cnt_parts):
    return pl.pallas_call(
        _tc_inv_body,
        out_shape=jax.ShapeDtypeStruct((1, NRP), jnp.float32),
    )(cnt_parts)


BN = 400
NBLK = N // BN


def _tc_xw_body(x_ref, w_ref, out_ref):
    out_ref[...] = jnp.dot(x_ref[...], w_ref[0],
                           preferred_element_type=jnp.float32)


def _tc_xw(x, w3):
    return pl.pallas_call(
        _tc_xw_body,
        grid=(R, NBLK),
        in_specs=[
            pl.BlockSpec((BN, G), lambda r, i: (i, 0)),
            pl.BlockSpec((1, G, H), lambda r, i: (r, 0, 0)),
        ],
        out_specs=pl.BlockSpec((BN, H), lambda r, i: (r * NBLK + i, 0)),
        out_shape=jax.ShapeDtypeStruct((R * N, H), jnp.float32),
    )(x, w3)


def _tc_h_body(parts_ref, x_ref, root_ref, bias_ref, out_ref):
    p = parts_ref[...]
    out_ref[...] = (p[0] + p[1]
                    + jnp.dot(x_ref[...], root_ref[...],
                              preferred_element_type=jnp.float32)
                    + bias_ref[...])


def _tc_h(parts, x, root, bias2):
    return pl.pallas_call(
        _tc_h_body,
        grid=(NBLK,),
        in_specs=[
            pl.BlockSpec((NC, BN, H), lambda i: (0, i, 0)),
            pl.BlockSpec((BN, G), lambda i: (i, 0)),
            pl.BlockSpec((G, H), lambda i: (0, 0)),
            pl.BlockSpec((1, H), lambda i: (0, 0)),
        ],
        out_specs=pl.BlockSpec((BN, H), lambda i: (i, 0)),
        out_shape=jax.ShapeDtypeStruct((N, H), jnp.float32),
    )(parts, x, root, bias2)


def _tc_out_body(parts_ref, h_ref, wrel_ref, brel_ref, wroot_ref, out_ref):
    p = parts_ref[...]
    out_ref[...] = (jnp.dot(p[0] + p[1], wrel_ref[...],
                            preferred_element_type=jnp.float32)
                    + brel_ref[...]
                    + jnp.dot(h_ref[...], wroot_ref[...],
                              preferred_element_type=jnp.float32))


def _tc_out(parts, h, wrel, brel2, wroot):
    return pl.pallas_call(
        _tc_out_body,
        grid=(NBLK,),
        in_specs=[
            pl.BlockSpec((NC, BN, H), lambda i: (0, i, 0)),
            pl.BlockSpec((BN, H), lambda i: (i, 0)),
            pl.BlockSpec((H, H), lambda i: (0, 0)),
            pl.BlockSpec((1, H), lambda i: (0, 0)),
            pl.BlockSpec((H, H), lambda i: (0, 0)),
        ],
        out_specs=pl.BlockSpec((BN, H), lambda i: (i, 0)),
        out_shape=jax.ShapeDtypeStruct((N, H), jnp.float32),
    )(parts, h, wrel, brel2, wroot)


def kernel(node_features, edge_index, edge_norm, edge_type, basis, comp,
           rgcn_root, rgcn_bias, gc_w_rel, gc_b_rel, gc_w_root):
    del edge_norm
    src = edge_index[0]
    dst = edge_index[1]
    padw = EPWD - ERW
    srcw = jnp.pad(src.reshape(NW, ERW), ((0, 0), (0, padw)))
    dstw = jnp.pad(dst.reshape(NW, ERW), ((0, 0), (0, padw)),
                   constant_values=N)
    typw = jnp.pad(edge_type.reshape(NW, ERW), ((0, 0), (0, padw)))
    edata = jnp.stack([srcw.reshape(NW, NCH2, CH),
                       dstw.reshape(NW, NCH2, CH),
                       typw.reshape(NW, NCH2, CH)], axis=2).reshape(-1)
    zeros_cnt = jnp.zeros((NRP,), jnp.float32)
    zeros_rows = jnp.zeros((NP, H), jnp.float32)
    ones_tpl = jnp.ones((CH,), jnp.float32)

    cnt_parts = _sc_counts(edata, zeros_cnt, ones_tpl).reshape(NC, NRP)
    inv = _tc_inv(cnt_parts).reshape(NRP)

    w3 = _tc_weight(comp, basis.reshape(NB, G * H)).reshape(R, G, H)
    xw = _tc_xw(node_features, w3)

    h_parts = _sc_rgcn(xw, inv, edata, zeros_rows)
    h = _tc_h(h_parts, node_features, rgcn_root, rgcn_bias.reshape(1, H))

    agg_parts = _sc_gconv(h, edata, zeros_rows)
    out = _tc_out(agg_parts, h, gc_w_rel, gc_b_rel.reshape(1, H), gc_w_root)
    return out
